# initial kernel scaffold (unmeasured)
import functools
import math

import jax
import jax.numpy as jnp
from jax import lax
from jax.experimental import pallas as pl
from jax.experimental.pallas import tpu as pltpu

N_DEV = 16
TQ = 512


def kernel(q, k, v):
    m_per, d = q.shape
    n_tiles = m_per // TQ
    scale = 1.0 / math.sqrt(d)

    def body(q_ref, k_ref, v_ref, out_ref, comm_ref, m_ref, l_ref,
             send_sems, recv_sems, credit_sems):
        my = lax.axis_index("i")
        left = (my - 1) % N_DEV
        right = (my + 1) % N_DEV

        barrier_sem = pltpu.get_barrier_semaphore()
        for nbr in (left, right):
            pl.semaphore_signal(
                barrier_sem, inc=1,
                device_id=(nbr,), device_id_type=pl.DeviceIdType.MESH,
            )
        pl.semaphore_wait(barrier_sem, 2)

        comm_ref[0, 0, :, :] = k_ref[:, :]
        comm_ref[0, 1, :, :] = v_ref[:, :]
        m_ref[:, :] = jnp.full_like(m_ref, -1e30)
        l_ref[:, :] = jnp.zeros_like(l_ref)
        out_ref[:, :] = jnp.zeros_like(out_ref)

        for h in range(N_DEV):
            cur = h % 2
            nxt = (h + 1) % 2

            rdma = None
            if h < N_DEV - 1:
                if h >= 2:
                    pl.semaphore_wait(credit_sems.at[nxt], 1)
                rdma = pltpu.make_async_remote_copy(
                    src_ref=comm_ref.at[cur],
                    dst_ref=comm_ref.at[nxt],
                    send_sem=send_sems.at[cur],
                    recv_sem=recv_sems.at[nxt],
                    device_id=(right,),
                    device_id_type=pl.DeviceIdType.MESH,
                )
                rdma.start()

            def tile_step(t, _):
                rows = pl.ds(t * TQ, TQ)
                qt = q_ref[rows, :]
                kb = comm_ref[cur, 0, :, :]
                vb = comm_ref[cur, 1, :, :]
                s = lax.dot_general(
                    qt, kb, (((1,), (1,)), ((), ())),
                    preferred_element_type=jnp.float32,
                ) * scale
                m_prev = m_ref[rows, :]
                m_new = jnp.maximum(m_prev, jnp.max(s, axis=1, keepdims=True))
                p = jnp.exp(s - m_new)
                alpha = jnp.exp(m_prev - m_new)
                l_new = l_ref[rows, :] * alpha + jnp.sum(
                    s.dtype.type(0) + p, axis=1, keepdims=True)
                acc = out_ref[rows, :] * alpha + jnp.dot(
                    p, vb, preferred_element_type=jnp.float32)
                m_ref[rows, :] = m_new
                l_ref[rows, :] = l_new
                out_ref[rows, :] = acc
                return 0

            lax.fori_loop(0, n_tiles, tile_step, 0)

            if rdma is not None:
                rdma.wait()
                pl.semaphore_signal(
                    credit_sems.at[cur], inc=1,
                    device_id=(left,), device_id_type=pl.DeviceIdType.MESH,
                )

        out_ref[:, :] = out_ref[:, :] / l_ref[:, :]

    return pl.pallas_call(
        body,
        out_shape=jax.ShapeDtypeStruct((m_per, d), jnp.float32),
        in_specs=[
            pl.BlockSpec(memory_space=pltpu.VMEM),
            pl.BlockSpec(memory_space=pltpu.VMEM),
            pl.BlockSpec(memory_space=pltpu.VMEM),
        ],
        out_specs=pl.BlockSpec(memory_space=pltpu.VMEM),
        scratch_shapes=[
            pltpu.VMEM((2, 2, m_per, d), jnp.float32),
            pltpu.VMEM((m_per, 1), jnp.float32),
            pltpu.VMEM((m_per, 1), jnp.float32),
            pltpu.SemaphoreType.DMA((2,)),
            pltpu.SemaphoreType.DMA((2,)),
            pltpu.SemaphoreType.REGULAR((2,)),
        ],
        compiler_params=pltpu.CompilerParams(collective_id=0),
    )(q, k, v)


# baseline (device time: 1456366 ns/iter reference)
import functools
import math

import jax
import jax.numpy as jnp
from jax import lax
from jax.experimental import pallas as pl
from jax.experimental.pallas import tpu as pltpu

N_DEV = 16
TQ = 512
_USE_CREDITS = True
_SERIALIZE = False


def kernel(q, k, v):
    m_per, d = q.shape
    n_tiles = m_per // TQ
    scale = 1.0 / math.sqrt(d)

    def body(q_ref, k_ref, v_ref, out_ref, comm_ref, m_ref, l_ref,
             send_sems, recv_sems, credit_sems):
        my = lax.axis_index("i")
        left = (my - 1) % N_DEV
        right = (my + 1) % N_DEV

        barrier_sem = pltpu.get_barrier_semaphore()
        for nbr in (left, right):
            pl.semaphore_signal(
                barrier_sem, inc=1,
                device_id=(nbr,), device_id_type=pl.DeviceIdType.MESH,
            )
        pl.semaphore_wait(barrier_sem, 2)

        comm_ref[0, 0, :, :] = k_ref[:, :]
        comm_ref[0, 1, :, :] = v_ref[:, :]
        m_ref[:, :] = jnp.full_like(m_ref, -1e30)
        l_ref[:, :] = jnp.zeros_like(l_ref)
        out_ref[:, :] = jnp.zeros_like(out_ref)

        for h in range(N_DEV):
            cur = h % 2
            nxt = (h + 1) % 2

            rdma = None
            if h < N_DEV - 1:
                if h >= 2 and _USE_CREDITS:
                    pl.semaphore_wait(credit_sems.at[nxt], 1)
                rdma = pltpu.make_async_remote_copy(
                    src_ref=comm_ref.at[cur],
                    dst_ref=comm_ref.at[nxt],
                    send_sem=send_sems.at[cur],
                    recv_sem=recv_sems.at[nxt],
                    device_id=(right,),
                    device_id_type=pl.DeviceIdType.MESH,
                )
                rdma.start()
                if _SERIALIZE:
                    rdma.wait()

            def tile_step(t, _):
                rows = pl.ds(t * TQ, TQ)
                qt = q_ref[rows, :]
                kb = comm_ref[cur, 0, :, :]
                vb = comm_ref[cur, 1, :, :]
                s = lax.dot_general(
                    qt, kb, (((1,), (1,)), ((), ())),
                    preferred_element_type=jnp.float32,
                ) * scale
                m_prev = m_ref[rows, :]
                m_new = jnp.maximum(m_prev, jnp.max(s, axis=1, keepdims=True))
                p = jnp.exp(s - m_new)
                alpha = jnp.exp(m_prev - m_new)
                l_new = l_ref[rows, :] * alpha + jnp.sum(
                    p, axis=1, keepdims=True)
                acc = out_ref[rows, :] * alpha + jnp.dot(
                    p, vb, preferred_element_type=jnp.float32)
                m_ref[rows, :] = m_new
                l_ref[rows, :] = l_new
                out_ref[rows, :] = acc
                return 0

            lax.fori_loop(0, n_tiles, tile_step, 0)

            if rdma is not None:
                if not _SERIALIZE:
                    rdma.wait()
                if _USE_CREDITS:
                    pl.semaphore_signal(
                        credit_sems.at[cur], inc=1,
                        device_id=(left,), device_id_type=pl.DeviceIdType.MESH,
                    )

        if _USE_CREDITS:
            pl.semaphore_wait(credit_sems.at[0], 2)

        out_ref[:, :] = out_ref[:, :] / l_ref[:, :]

    return pl.pallas_call(
        body,
        out_shape=jax.ShapeDtypeStruct((m_per, d), jnp.float32),
        in_specs=[
            pl.BlockSpec(memory_space=pltpu.VMEM),
            pl.BlockSpec(memory_space=pltpu.VMEM),
            pl.BlockSpec(memory_space=pltpu.VMEM),
        ],
        out_specs=pl.BlockSpec(memory_space=pltpu.VMEM),
        scratch_shapes=[
            pltpu.VMEM((2, 2, m_per, d), jnp.float32),
            pltpu.VMEM((m_per, 1), jnp.float32),
            pltpu.VMEM((m_per, 1), jnp.float32),
            pltpu.SemaphoreType.DMA((2,)),
            pltpu.SemaphoreType.DMA((2,)),
            pltpu.SemaphoreType.REGULAR((2,)),
        ],
        compiler_params=pltpu.CompilerParams(collective_id=0),
    )(q, k, v)
